# baseline (speedup 1.0000x reference)
import functools

import jax
import jax.numpy as jnp
from jax import lax
from jax.experimental import pallas as pl
from jax.experimental.pallas import tpu as pltpu
from jax.experimental.pallas import tpu_sc as plsc

NUM_CORES = 2
NUM_SUBCORES = 16
NW = NUM_CORES * NUM_SUBCORES
SCHUNK = 4
DEPTH = 2
LANES = 16


def _sc_body(x_hbm, emb_hbm, out_hbm, xbuf, ebuf, obuf, xsem, esem, osem):
    batch, seq_len, dim = x_hbm.shape
    off = emb_hbm.shape[0] - seq_len
    per_w = seq_len // NW
    n = per_w // SCHUNK
    wid = lax.axis_index("s") * NUM_CORES + lax.axis_index("c")
    base = wid * per_w

    def in_copies(i, slot):
        s0 = base + i * SCHUNK
        cps = [pltpu.make_async_copy(
            emb_hbm.at[pl.ds(off + s0, SCHUNK), :], ebuf.at[slot], esem.at[slot])]
        for b in range(batch):
            cps.append(pltpu.make_async_copy(
                x_hbm.at[b, pl.ds(s0, SCHUNK), :], xbuf.at[slot, b], xsem.at[slot]))
        return cps

    def out_copies(i, slot):
        s0 = base + i * SCHUNK
        return [pltpu.make_async_copy(
            obuf.at[slot, b], out_hbm.at[b, pl.ds(s0, SCHUNK), :], osem.at[slot])
            for b in range(batch)]

    for s in range(DEPTH):
        for c in in_copies(s, s):
            c.start()

    def body(i, carry):
        slot = lax.rem(i, DEPTH)
        for c in in_copies(i, slot):
            c.wait()

        @pl.when(i >= DEPTH)
        def _wait_out():
            for c in out_copies(i - DEPTH, slot):
                c.wait()

        def col(cidx, carry2):
            d0 = cidx * LANES
            for s in range(SCHUNK):
                e = ebuf[slot, s, pl.ds(d0, LANES)]
                for b in range(batch):
                    obuf[slot, b, s, pl.ds(d0, LANES)] = (
                        xbuf[slot, b, s, pl.ds(d0, LANES)] + e)
            return carry2

        lax.fori_loop(0, dim // LANES, col, 0)

        for c in out_copies(i, slot):
            c.start()

        @pl.when(i + DEPTH < n)
        def _next_in():
            for c in in_copies(i + DEPTH, slot):
                c.start()

        return carry

    lax.fori_loop(0, n, body, 0)

    for k in range(max(0, n - DEPTH), n):
        for c in out_copies(k, k % DEPTH):
            c.wait()


def kernel(x, emb_weight):
    batch, seq_len, dim = x.shape
    run = functools.partial(
        pl.kernel,
        out_type=jax.ShapeDtypeStruct(x.shape, x.dtype),
        mesh=plsc.VectorSubcoreMesh(
            core_axis_name="c", subcore_axis_name="s",
            num_cores=NUM_CORES, num_subcores=NUM_SUBCORES),
        scratch_types=[
            pltpu.VMEM((DEPTH, batch, SCHUNK, dim), jnp.float32),
            pltpu.VMEM((DEPTH, SCHUNK, dim), jnp.float32),
            pltpu.VMEM((DEPTH, batch, SCHUNK, dim), jnp.float32),
            pltpu.SemaphoreType.DMA((DEPTH,)),
            pltpu.SemaphoreType.DMA((DEPTH,)),
            pltpu.SemaphoreType.DMA((DEPTH,)),
        ],
    )(_sc_body)
    return run(x, emb_weight)


# speedup vs baseline: 2.4113x; 2.4113x over previous
import functools

import jax
import jax.numpy as jnp
from jax import lax
from jax.experimental import pallas as pl
from jax.experimental.pallas import tpu as pltpu
from jax.experimental.pallas import tpu_sc as plsc

NUM_CORES = 2
NUM_SUBCORES = 16
NW = NUM_CORES * NUM_SUBCORES
SCHUNK = 4
DEPTH = 2
LANES = 16
UNROLL = 4


def _sc_body(x_hbm, emb_hbm, out_hbm, xbuf, ebuf, obuf, xsem, esem, osem):
    batch, seq_len, dim = x_hbm.shape
    off = emb_hbm.shape[0] - seq_len
    per_w = seq_len // NW
    n = per_w // SCHUNK
    wid = lax.axis_index("s") * NUM_CORES + lax.axis_index("c")
    base = wid * per_w

    def in_copies(i, slot):
        s0 = base + i * SCHUNK
        cps = [pltpu.make_async_copy(
            emb_hbm.at[pl.ds(off + s0, SCHUNK), :], ebuf.at[slot], esem.at[slot])]
        for b in range(batch):
            cps.append(pltpu.make_async_copy(
                x_hbm.at[b, pl.ds(s0, SCHUNK), :], xbuf.at[slot, b], xsem.at[slot]))
        return cps

    def out_copies(i, slot):
        s0 = base + i * SCHUNK
        return [pltpu.make_async_copy(
            obuf.at[slot, b], out_hbm.at[b, pl.ds(s0, SCHUNK), :], osem.at[slot])
            for b in range(batch)]

    for s in range(DEPTH):
        for c in in_copies(s, s):
            c.start()

    def group(g, carry):
        for slot in range(DEPTH):
            i = g * DEPTH + slot
            for c in in_copies(i, slot):
                c.wait()

            @pl.when(i >= DEPTH)
            def _wait_out(i=i, slot=slot):
                for c in out_copies(i - DEPTH, slot):
                    c.wait()

            @plsc.parallel_loop(0, dim, LANES, unroll=UNROLL)
            def _col(d0, slot=slot):
                for s in range(SCHUNK):
                    e = ebuf[slot, s, pl.ds(d0, LANES)]
                    for b in range(batch):
                        obuf[slot, b, s, pl.ds(d0, LANES)] = (
                            xbuf[slot, b, s, pl.ds(d0, LANES)] + e)

            for c in out_copies(i, slot):
                c.start()

            @pl.when(i + DEPTH < n)
            def _next_in(i=i, slot=slot):
                for c in in_copies(i + DEPTH, slot):
                    c.start()

        return carry

    lax.fori_loop(0, n // DEPTH, group, 0)

    for k in range(max(0, n - DEPTH), n):
        for c in out_copies(k, k % DEPTH):
            c.wait()


def kernel(x, emb_weight):
    batch, seq_len, dim = x.shape
    run = functools.partial(
        pl.kernel,
        out_type=jax.ShapeDtypeStruct(x.shape, x.dtype),
        mesh=plsc.VectorSubcoreMesh(
            core_axis_name="c", subcore_axis_name="s",
            num_cores=NUM_CORES, num_subcores=NUM_SUBCORES),
        scratch_types=[
            pltpu.VMEM((DEPTH, batch, SCHUNK, dim), jnp.float32),
            pltpu.VMEM((DEPTH, SCHUNK, dim), jnp.float32),
            pltpu.VMEM((DEPTH, batch, SCHUNK, dim), jnp.float32),
            pltpu.SemaphoreType.DMA((DEPTH,)),
            pltpu.SemaphoreType.DMA((DEPTH,)),
            pltpu.SemaphoreType.DMA((DEPTH,)),
        ],
    )(_sc_body)
    return run(x, emb_weight)
